# baseline (speedup 1.0000x reference)
import functools

import jax
import jax.numpy as jnp
from jax import lax
from jax.experimental import pallas as pl
from jax.experimental.pallas import tpu as pltpu
from jax.experimental.pallas import tpu_sc as plsc

_B = 4096
_M = 10
_N = 1000
_NC = 2
_NS = 16
_NW = _NC * _NS
_L = 16
_BPW = _B // _NW
_NCH = _BPW // _L
_BB = 256
_NB = _B // _BB


def _tables_body(p1ab_ref, p2ab_ref, p1ba_ref, p2ba_ref, cn_ref, nrm_ref):
    for k, (p1, p2) in enumerate(((p1ab_ref, p2ab_ref), (p1ba_ref, p2ba_ref))):
        t = p2[:, :]
        m = jnp.max(t, axis=0)
        s = jnp.sum(jnp.exp(t - m[None, :]), axis=0)
        cn_ref[k, :] = jnp.log(s) + m
        v = p1[:]
        mv = jnp.max(v)
        nrm_ref[k] = jnp.log(jnp.sum(jnp.exp(v - mv))) + mv


def _argmax_body(x_ref, oa_ref, ob_ref):
    oa_ref[...] = jnp.sum(x_ref[:, 0, :], axis=-1).astype(jnp.int32) % _N
    ob_ref[...] = jnp.sum(x_ref[:, 1, :], axis=-1).astype(jnp.int32) % _N


def _sc_body(idxa_hbm, idxb_hbm, p1ab_hbm, cnab_hbm, p1ba_hbm, cnba_hbm,
             p2ab_hbm, p2ba_hbm, out_hbm,
             idx_a, idx_b, fidx, gv, g1, g2, out_v, sem0, sem1):
    wid = lax.axis_index("s") * _NC + lax.axis_index("c")
    base = wid * _BPW

    ha = pltpu.async_copy(idxa_hbm.at[pl.ds(base, _BPW)], idx_a, sem0)
    hb = pltpu.async_copy(idxb_hbm.at[pl.ds(base, _BPW)], idx_b, sem1)
    ha.wait()
    hb.wait()

    for model, (idx1, p1_hbm, cn_hbm, p2_hbm) in enumerate((
            (idx_a, p1ab_hbm, cnab_hbm, p2ab_hbm),
            (idx_b, p1ba_hbm, cnba_hbm, p2ba_hbm))):
        for c in range(_NCH):
            va = idx_a[pl.ds(c * _L, _L)]
            vb = idx_b[pl.ds(c * _L, _L)]
            f = vb * _N + va if model == 0 else va * _N + vb
            fidx[pl.ds(c * _L, _L)] = f
        h2 = pltpu.async_copy(p2_hbm.at[fidx], gv, sem0)
        hp = pltpu.async_copy(p1_hbm.at[idx1], g1, sem0)
        hc = pltpu.async_copy(cn_hbm.at[idx1], g2, sem0)
        h2.wait()
        hp.wait()
        hc.wait()
        acc = jnp.zeros((_L,), jnp.float32)
        for c in range(_NCH):
            s = pl.ds(c * _L, _L)
            acc = acc + gv[s] + g1[s] - g2[s]
        out_v[model, :] = acc

    pltpu.sync_copy(out_v, out_hbm.at[wid])


@functools.cache
def _make_sc_kernel():
    mesh = plsc.VectorSubcoreMesh(core_axis_name="c", subcore_axis_name="s",
                                  num_cores=_NC, num_subcores=_NS)
    return pl.kernel(
        _sc_body,
        mesh=mesh,
        compiler_params=pltpu.CompilerParams(needs_layout_passes=False),
        out_type=jax.ShapeDtypeStruct((_NW, 2, _L), jnp.float32),
        scratch_types=[
            pltpu.VMEM((_BPW,), jnp.int32),
            pltpu.VMEM((_BPW,), jnp.int32),
            pltpu.VMEM((_BPW,), jnp.int32),
            pltpu.VMEM((_BPW,), jnp.float32),
            pltpu.VMEM((_BPW,), jnp.float32),
            pltpu.VMEM((_BPW,), jnp.float32),
            pltpu.VMEM((2, _L), jnp.float32),
            pltpu.SemaphoreType.DMA,
            pltpu.SemaphoreType.DMA,
        ],
    )


def kernel(samples, P_1_AB, P_2_1_AB, P_1_BA, P_2_1_BA, gamma):
    B, M, N = samples.shape

    cn, nrm = pl.pallas_call(
        _tables_body,
        in_specs=[
            pl.BlockSpec((N,), lambda: (0,)),
            pl.BlockSpec((N, N), lambda: (0, 0)),
            pl.BlockSpec((N,), lambda: (0,)),
            pl.BlockSpec((N, N), lambda: (0, 0)),
        ],
        out_specs=[
            pl.BlockSpec((2, N), lambda: (0, 0)),
            pl.BlockSpec(memory_space=pltpu.SMEM),
        ],
        out_shape=[
            jax.ShapeDtypeStruct((2, N), jnp.float32),
            jax.ShapeDtypeStruct((2,), jnp.float32),
        ],
    )(P_1_AB, P_2_1_AB, P_1_BA, P_2_1_BA)

    idx_a, idx_b = pl.pallas_call(
        _argmax_body,
        grid=(_NB,),
        in_specs=[pl.BlockSpec((_BB, _M, N), lambda i: (i, 0, 0))],
        out_specs=[
            pl.BlockSpec((_BB,), lambda i: (i,)),
            pl.BlockSpec((_BB,), lambda i: (i,)),
        ],
        out_shape=[
            jax.ShapeDtypeStruct((B,), jnp.int32),
            jax.ShapeDtypeStruct((B,), jnp.int32),
        ],
    )(samples)

    partials = _make_sc_kernel()(
        idx_a, idx_b, P_1_AB, cn[0], P_1_BA, cn[1],
        P_2_1_AB.reshape(-1), P_2_1_BA.reshape(-1))
    sums = jnp.sum(partials, axis=(0, 2))

    log_w = gamma - jax.scipy.special.logsumexp(gamma)
    m_ab = log_w[0] + sums[0] - B * nrm[0]
    m_ba = log_w[1] + sums[1] - B * nrm[1]
    return jnp.logaddexp(m_ab, m_ba)


# speedup vs baseline: 1.0402x; 1.0402x over previous
import functools

import jax
import jax.numpy as jnp
from jax import lax
from jax.experimental import pallas as pl
from jax.experimental.pallas import tpu as pltpu
from jax.experimental.pallas import tpu_sc as plsc

_B = 4096
_M = 10
_N = 1000
_NC = 2
_NS = 16
_NW = _NC * _NS
_L = 16
_BPW = _B // _NW
_NCH = _BPW // _L
_BB = 256
_NB = _B // _BB


def _tables_body(p1ab_ref, p2ab_ref, p1ba_ref, p2ba_ref, cn_ref, nrm_ref):
    for k, (p1, p2) in enumerate(((p1ab_ref, p2ab_ref), (p1ba_ref, p2ba_ref))):
        t = p2[:, :]
        m = jnp.max(t, axis=0)
        s = jnp.sum(jnp.exp(t - m[None, :]), axis=0)
        cn_ref[k, :] = jnp.log(s) + m
        v = p1[:]
        mv = jnp.max(v)
        nrm_ref[k] = jnp.log(jnp.sum(jnp.exp(v - mv))) + mv


def _argmax_body(x_ref, oa_ref, ob_ref):
    oa_ref[...] = jnp.argmax(x_ref[:, 0, :], axis=-1).astype(jnp.int32)
    ob_ref[...] = jnp.argmax(x_ref[:, 1, :], axis=-1).astype(jnp.int32)


def _sc_body(idxa_hbm, idxb_hbm, p1ab_hbm, cnab_hbm, p1ba_hbm, cnba_hbm,
             p2ab_hbm, p2ba_hbm, out_hbm,
             idx_a, idx_b, fidx, gv, g1, g2, out_v, sem0, sem1):
    wid = lax.axis_index("s") * _NC + lax.axis_index("c")
    base = wid * _BPW

    ha = pltpu.async_copy(idxa_hbm.at[pl.ds(base, _BPW)], idx_a, sem0)
    hb = pltpu.async_copy(idxb_hbm.at[pl.ds(base, _BPW)], idx_b, sem1)
    ha.wait()
    hb.wait()

    for model, (idx1, p1_hbm, cn_hbm, p2_hbm) in enumerate((
            (idx_a, p1ab_hbm, cnab_hbm, p2ab_hbm),
            (idx_b, p1ba_hbm, cnba_hbm, p2ba_hbm))):
        for c in range(_NCH):
            va = idx_a[pl.ds(c * _L, _L)]
            vb = idx_b[pl.ds(c * _L, _L)]
            f = vb * _N + va if model == 0 else va * _N + vb
            fidx[pl.ds(c * _L, _L)] = f
        h2 = pltpu.async_copy(p2_hbm.at[fidx], gv, sem0)
        hp = pltpu.async_copy(p1_hbm.at[idx1], g1, sem0)
        hc = pltpu.async_copy(cn_hbm.at[idx1], g2, sem0)
        h2.wait()
        hp.wait()
        hc.wait()
        acc = jnp.zeros((_L,), jnp.float32)
        for c in range(_NCH):
            s = pl.ds(c * _L, _L)
            acc = acc + gv[s] + g1[s] - g2[s]
        out_v[model, :] = acc

    pltpu.sync_copy(out_v, out_hbm.at[wid])


@functools.cache
def _make_sc_kernel():
    mesh = plsc.VectorSubcoreMesh(core_axis_name="c", subcore_axis_name="s",
                                  num_cores=_NC, num_subcores=_NS)
    return pl.kernel(
        _sc_body,
        mesh=mesh,
        compiler_params=pltpu.CompilerParams(needs_layout_passes=False),
        out_type=jax.ShapeDtypeStruct((_NW, 2, _L), jnp.float32),
        scratch_types=[
            pltpu.VMEM((_BPW,), jnp.int32),
            pltpu.VMEM((_BPW,), jnp.int32),
            pltpu.VMEM((_BPW,), jnp.int32),
            pltpu.VMEM((_BPW,), jnp.float32),
            pltpu.VMEM((_BPW,), jnp.float32),
            pltpu.VMEM((_BPW,), jnp.float32),
            pltpu.VMEM((2, _L), jnp.float32),
            pltpu.SemaphoreType.DMA,
            pltpu.SemaphoreType.DMA,
        ],
    )


def kernel(samples, P_1_AB, P_2_1_AB, P_1_BA, P_2_1_BA, gamma):
    B, M, N = samples.shape

    cn, nrm = pl.pallas_call(
        _tables_body,
        in_specs=[
            pl.BlockSpec((N,), lambda: (0,)),
            pl.BlockSpec((N, N), lambda: (0, 0)),
            pl.BlockSpec((N,), lambda: (0,)),
            pl.BlockSpec((N, N), lambda: (0, 0)),
        ],
        out_specs=[
            pl.BlockSpec((2, N), lambda: (0, 0)),
            pl.BlockSpec(memory_space=pltpu.SMEM),
        ],
        out_shape=[
            jax.ShapeDtypeStruct((2, N), jnp.float32),
            jax.ShapeDtypeStruct((2,), jnp.float32),
        ],
    )(P_1_AB, P_2_1_AB, P_1_BA, P_2_1_BA)

    idx_a, idx_b = pl.pallas_call(
        _argmax_body,
        grid=(_NB,),
        in_specs=[pl.BlockSpec((_BB, _M, N), lambda i: (i, 0, 0))],
        out_specs=[
            pl.BlockSpec((_BB,), lambda i: (i,)),
            pl.BlockSpec((_BB,), lambda i: (i,)),
        ],
        out_shape=[
            jax.ShapeDtypeStruct((B,), jnp.int32),
            jax.ShapeDtypeStruct((B,), jnp.int32),
        ],
    )(samples)

    partials = _make_sc_kernel()(
        idx_a, idx_b, P_1_AB, cn[0], P_1_BA, cn[1],
        P_2_1_AB.reshape(-1), P_2_1_BA.reshape(-1))
    sums = jnp.sum(partials, axis=(0, 2))

    log_w = gamma - jax.scipy.special.logsumexp(gamma)
    m_ab = log_w[0] + sums[0] - B * nrm[0]
    m_ba = log_w[1] + sums[1] - B * nrm[1]
    return jnp.logaddexp(m_ab, m_ba)


# speedup vs baseline: 1.1995x; 1.1531x over previous
import functools

import jax
import jax.numpy as jnp
from jax import lax
from jax.experimental import pallas as pl
from jax.experimental.pallas import tpu as pltpu
from jax.experimental.pallas import tpu_sc as plsc

_B = 4096
_M = 10
_N = 1000
_NC = 2
_NS = 16
_NW = _NC * _NS
_L = 16
_BPW = _B // _NW
_NCH = _BPW // _L
_CH = 128
_NCK = _B // _CH
_K = 6


def _tables_body(p1ab_ref, p2ab_ref, p1ba_ref, p2ba_ref, cn_ref, nrm_ref):
    for k, (p1, p2) in enumerate(((p1ab_ref, p2ab_ref), (p1ba_ref, p2ba_ref))):
        t = p2[:, :]
        m = jnp.max(t, axis=0)
        s = jnp.sum(jnp.exp(t - m[None, :]), axis=0)
        cn_ref[k, :] = jnp.log(s) + m
        v = p1[:]
        mv = jnp.max(v)
        nrm_ref[k] = jnp.log(jnp.sum(jnp.exp(v - mv))) + mv


def _argmax_body(s_hbm, oa_ref, ob_ref, buf, sems):
    i = pl.program_id(0)

    def _copy(chunk, slot, row):
        return pltpu.make_async_copy(
            s_hbm.at[pl.ds(chunk * _CH, _CH), row, :],
            buf.at[slot, row],
            sems.at[slot, row])

    @pl.when(i == 0)
    def _warmup():
        for c in range(_K):
            for r in range(2):
                _copy(c, c, r).start()

    slot = lax.rem(i, _K)
    for r in range(2):
        _copy(i, slot, r).wait()
    oa_ref[pl.ds(i * _CH, _CH)] = jnp.argmax(buf[slot, 0], axis=-1).astype(jnp.int32)
    ob_ref[pl.ds(i * _CH, _CH)] = jnp.argmax(buf[slot, 1], axis=-1).astype(jnp.int32)

    nxt = i + _K

    @pl.when(nxt < _NCK)
    def _refill():
        for r in range(2):
            _copy(nxt, slot, r).start()


def _sc_body(idxa_hbm, idxb_hbm, p1ab_hbm, cnab_hbm, p1ba_hbm, cnba_hbm,
             p2ab_hbm, p2ba_hbm, out_hbm,
             idx_a, idx_b, fidx, gv, g1, g2, out_v, sem0, sem1):
    wid = lax.axis_index("s") * _NC + lax.axis_index("c")
    base = wid * _BPW

    ha = pltpu.async_copy(idxa_hbm.at[pl.ds(base, _BPW)], idx_a, sem0)
    hb = pltpu.async_copy(idxb_hbm.at[pl.ds(base, _BPW)], idx_b, sem1)
    ha.wait()
    hb.wait()

    for model, (idx1, p1_hbm, cn_hbm, p2_hbm) in enumerate((
            (idx_a, p1ab_hbm, cnab_hbm, p2ab_hbm),
            (idx_b, p1ba_hbm, cnba_hbm, p2ba_hbm))):
        for c in range(_NCH):
            va = idx_a[pl.ds(c * _L, _L)]
            vb = idx_b[pl.ds(c * _L, _L)]
            f = vb * _N + va if model == 0 else va * _N + vb
            fidx[pl.ds(c * _L, _L)] = f
        h2 = pltpu.async_copy(p2_hbm.at[fidx], gv, sem0)
        hp = pltpu.async_copy(p1_hbm.at[idx1], g1, sem0)
        hc = pltpu.async_copy(cn_hbm.at[idx1], g2, sem0)
        h2.wait()
        hp.wait()
        hc.wait()
        acc = jnp.zeros((_L,), jnp.float32)
        for c in range(_NCH):
            s = pl.ds(c * _L, _L)
            acc = acc + gv[s] + g1[s] - g2[s]
        out_v[model, :] = acc

    pltpu.sync_copy(out_v, out_hbm.at[wid])


@functools.cache
def _make_sc_kernel():
    mesh = plsc.VectorSubcoreMesh(core_axis_name="c", subcore_axis_name="s",
                                  num_cores=_NC, num_subcores=_NS)
    return pl.kernel(
        _sc_body,
        mesh=mesh,
        compiler_params=pltpu.CompilerParams(needs_layout_passes=False),
        out_type=jax.ShapeDtypeStruct((_NW, 2, _L), jnp.float32),
        scratch_types=[
            pltpu.VMEM((_BPW,), jnp.int32),
            pltpu.VMEM((_BPW,), jnp.int32),
            pltpu.VMEM((_BPW,), jnp.int32),
            pltpu.VMEM((_BPW,), jnp.float32),
            pltpu.VMEM((_BPW,), jnp.float32),
            pltpu.VMEM((_BPW,), jnp.float32),
            pltpu.VMEM((2, _L), jnp.float32),
            pltpu.SemaphoreType.DMA,
            pltpu.SemaphoreType.DMA,
        ],
    )


def kernel(samples, P_1_AB, P_2_1_AB, P_1_BA, P_2_1_BA, gamma):
    B, M, N = samples.shape

    cn, nrm = pl.pallas_call(
        _tables_body,
        in_specs=[
            pl.BlockSpec((N,), lambda: (0,)),
            pl.BlockSpec((N, N), lambda: (0, 0)),
            pl.BlockSpec((N,), lambda: (0,)),
            pl.BlockSpec((N, N), lambda: (0, 0)),
        ],
        out_specs=[
            pl.BlockSpec((2, N), lambda: (0, 0)),
            pl.BlockSpec(memory_space=pltpu.SMEM),
        ],
        out_shape=[
            jax.ShapeDtypeStruct((2, N), jnp.float32),
            jax.ShapeDtypeStruct((2,), jnp.float32),
        ],
    )(P_1_AB, P_2_1_AB, P_1_BA, P_2_1_BA)

    idx_a, idx_b = pl.pallas_call(
        _argmax_body,
        grid=(_NCK,),
        in_specs=[pl.BlockSpec(memory_space=pl.ANY)],
        out_specs=[
            pl.BlockSpec((B,), lambda i: (0,)),
            pl.BlockSpec((B,), lambda i: (0,)),
        ],
        out_shape=[
            jax.ShapeDtypeStruct((B,), jnp.int32),
            jax.ShapeDtypeStruct((B,), jnp.int32),
        ],
        scratch_shapes=[
            pltpu.VMEM((_K, 2, _CH, N), jnp.float32),
            pltpu.SemaphoreType.DMA((_K, 2)),
        ],
    )(samples)

    partials = _make_sc_kernel()(
        idx_a, idx_b, P_1_AB, cn[0], P_1_BA, cn[1],
        P_2_1_AB.reshape(-1), P_2_1_BA.reshape(-1))
    sums = jnp.sum(partials, axis=(0, 2))

    log_w = gamma - jax.scipy.special.logsumexp(gamma)
    m_ab = log_w[0] + sums[0] - B * nrm[0]
    m_ba = log_w[1] + sums[1] - B * nrm[1]
    return jnp.logaddexp(m_ab, m_ba)
